# baseline (speedup 1.0000x reference)
import functools

import jax
import jax.numpy as jnp
from jax import lax
from jax.experimental import pallas as pl
from jax.experimental.pallas import tpu as pltpu
from jax.experimental.pallas import tpu_sc as plsc

_N = 50000
_VOCAB = 1000
_HIDDEN = 128


_ROWS_PER_BLK = 5000
_NBLK = _N // _ROWS_PER_BLK


def _argmax_body(x_ref, o_ref):
    x = x_ref[...]
    m = jnp.max(x, axis=1, keepdims=True)
    cols = lax.broadcasted_iota(jnp.int32, x.shape, 1)
    idx = jnp.min(jnp.where(x == m, cols, _VOCAB), axis=1)
    o_ref[...] = idx.reshape(1, 1, _ROWS_PER_BLK)


def _argmax_tc(x, nblk):
    return pl.pallas_call(
        _argmax_body,
        grid=(nblk,),
        in_specs=[pl.BlockSpec((_ROWS_PER_BLK, _VOCAB), lambda i: (i, 0))],
        out_specs=pl.BlockSpec((1, 1, _ROWS_PER_BLK), lambda i: (i, 0, 0)),
        out_shape=jax.ShapeDtypeStruct((nblk, 1, _ROWS_PER_BLK), jnp.int32),
        compiler_params=pltpu.CompilerParams(
            dimension_semantics=("arbitrary",),
        ),
    )(x)


_K_SC = 15000
_SC_BASE = _N - _K_SC
_RC = 40
_SC_CHUNKS = _K_SC // _RC
_NVEC = _VOCAB // 16


@functools.cache
def _make_argmax_sc():
    mesh = plsc.VectorSubcoreMesh(
        core_axis_name="c", subcore_axis_name="s", num_cores=2, num_subcores=16
    )
    nw = mesh.num_cores * mesh.num_subcores

    @functools.partial(
        pl.kernel,
        out_type=jax.ShapeDtypeStruct((_K_SC,), jnp.int32),
        mesh=mesh,
        scratch_types=[
            pltpu.VMEM((_RC, _VOCAB), jnp.float32),
            pltpu.VMEM((_RC,), jnp.int32),
        ],
        compiler_params=pltpu.CompilerParams(
            use_tc_tiling_on_sc=True, needs_layout_passes=False
        ),
    )
    def _amax_sc(x_hbm, idx_hbm, buf, idxb):
        wid = lax.axis_index("s") * mesh.num_cores + lax.axis_index("c")
        n_chunks = (_SC_CHUNKS - wid + nw - 1) // nw
        lane = lax.iota(jnp.int32, 16)

        def chunk_body(i, carry):
            t = wid + i * nw
            a = pl.multiple_of(_SC_BASE + t * _RC, 8)
            pltpu.sync_copy(x_hbm.at[pl.ds(a, _RC)], buf)

            def row_body(r, c2):
                m = buf[r, pl.ds(0, 16)]
                cb = lane

                def col_body(j, mc):
                    m, cb = mc
                    v = buf[r, pl.ds(j * 16, 16)]
                    cmp = v > m
                    col = lane + j * 16
                    return (jnp.where(cmp, v, m), jnp.where(cmp, col, cb))

                m, cb = lax.fori_loop(1, _NVEC, col_body, (m, cb))
                v = buf[r, pl.ds(_VOCAB - 16, 16)]
                cmp = v > m
                m = jnp.where(cmp, v, m)
                cb = jnp.where(cmp, lane + (_VOCAB - 16), cb)
                big = jnp.max(m)
                cand = jnp.where(m == big, cb, jnp.int32(2**30))
                mn = jnp.min(cand)
                plsc.store_scatter(
                    idxb,
                    [jnp.full((16,), r, jnp.int32)],
                    cand,
                    mask=cand == mn,
                )
                return c2

            lax.fori_loop(0, _RC, row_body, 0)
            pltpu.sync_copy(idxb, idx_hbm.at[pl.ds(t * _RC, _RC)])
            return carry

        lax.fori_loop(0, n_chunks, chunk_body, 0)

    return _amax_sc


_C = 112
_TOTAL_CHUNKS = -(-_N // _C)
_LAST_OFF = _N - _C


@functools.cache
def _make_gather_sc():
    mesh = plsc.VectorSubcoreMesh(
        core_axis_name="c", subcore_axis_name="s", num_cores=2, num_subcores=16
    )
    nw = mesh.num_cores * mesh.num_subcores

    @functools.partial(
        pl.kernel,
        out_type=jax.ShapeDtypeStruct((_N, _HIDDEN), jnp.float32),
        mesh=mesh,
        scratch_types=[
            pltpu.VMEM((_C,), jnp.int32),
            pltpu.VMEM((_C, _HIDDEN), jnp.float32),
            pltpu.SemaphoreType.DMA,
        ],
    )
    def _gather_sc(table_hbm, idx_hbm, out_hbm, idx_v, rows_v, sem):
        wid = lax.axis_index("s") * mesh.num_cores + lax.axis_index("c")
        n_chunks = (_TOTAL_CHUNKS - wid + nw - 1) // nw

        def body(i, carry):
            t = wid + i * nw
            off = jnp.minimum(t * _C, _LAST_OFF)
            off = pl.multiple_of(off, 8)
            pltpu.sync_copy(idx_hbm.at[pl.ds(off, _C)], idx_v)
            pltpu.async_copy(table_hbm.at[idx_v], rows_v, sem).wait()
            pltpu.sync_copy(rows_v, out_hbm.at[pl.ds(off, _C)])
            return carry

        lax.fori_loop(0, n_chunks, body, 0)

    return _gather_sc


def kernel(clique_attr, emb_weight):
    idx_sc = _make_argmax_sc()(clique_attr)
    idx_tc = _argmax_tc(clique_attr, _SC_BASE // _ROWS_PER_BLK).reshape(_SC_BASE)
    idx = jnp.concatenate([idx_tc, idx_sc])
    return _make_gather_sc()(emb_weight, idx)


# speedup vs baseline: 1.1214x; 1.1214x over previous
import functools

import jax
import jax.numpy as jnp
from jax import lax
from jax.experimental import pallas as pl
from jax.experimental.pallas import tpu as pltpu
from jax.experimental.pallas import tpu_sc as plsc

_N = 50000
_VOCAB = 1000
_HIDDEN = 128


_ROWS_PER_BLK = 5000
_NBLK = _N // _ROWS_PER_BLK


def _argmax_body(x_ref, o_ref):
    x = x_ref[...]
    m = jnp.max(x, axis=1, keepdims=True)
    cols = lax.broadcasted_iota(jnp.int32, x.shape, 1)
    idx = jnp.min(jnp.where(x == m, cols, _VOCAB), axis=1)
    o_ref[...] = idx.reshape(1, 1, _ROWS_PER_BLK)


def _argmax_tc(x, nblk):
    return pl.pallas_call(
        _argmax_body,
        grid=(nblk,),
        in_specs=[pl.BlockSpec((_ROWS_PER_BLK, _VOCAB), lambda i: (i, 0))],
        out_specs=pl.BlockSpec((1, 1, _ROWS_PER_BLK), lambda i: (i, 0, 0)),
        out_shape=jax.ShapeDtypeStruct((nblk, 1, _ROWS_PER_BLK), jnp.int32),
        compiler_params=pltpu.CompilerParams(
            dimension_semantics=("arbitrary",),
        ),
    )(x)


_K_SC = 15000
_SC_BASE = _N - _K_SC
_RC = 40
_SC_CHUNKS = _K_SC // _RC
_NVEC = _VOCAB // 16


@functools.cache
def _make_argmax_sc():
    mesh = plsc.VectorSubcoreMesh(
        core_axis_name="c", subcore_axis_name="s", num_cores=2, num_subcores=16
    )
    nw = mesh.num_cores * mesh.num_subcores

    @functools.partial(
        pl.kernel,
        out_type=jax.ShapeDtypeStruct((_K_SC,), jnp.int32),
        mesh=mesh,
        scratch_types=[
            pltpu.VMEM((_RC, _VOCAB), jnp.float32),
            pltpu.VMEM((_RC,), jnp.int32),
        ],
        compiler_params=pltpu.CompilerParams(
            use_tc_tiling_on_sc=True, needs_layout_passes=False
        ),
    )
    def _amax_sc(x_hbm, idx_hbm, buf, idxb):
        wid = lax.axis_index("s") * mesh.num_cores + lax.axis_index("c")
        n_chunks = (_SC_CHUNKS - wid + nw - 1) // nw
        lane = lax.iota(jnp.int32, 16)

        def chunk_body(i, carry):
            t = wid + i * nw
            a = pl.multiple_of(_SC_BASE + t * _RC, 8)
            pltpu.sync_copy(x_hbm.at[pl.ds(a, _RC)], buf)

            def row_body(r, c2):
                m = buf[r, pl.ds(0, 16)]
                cb = lane

                for j in range(1, _NVEC):
                    v = buf[r, pl.ds(j * 16, 16)]
                    cmp = v > m
                    m = jnp.where(cmp, v, m)
                    cb = jnp.where(cmp, lane + j * 16, cb)
                v = buf[r, pl.ds(_VOCAB - 16, 16)]
                cmp = v > m
                m = jnp.where(cmp, v, m)
                cb = jnp.where(cmp, lane + (_VOCAB - 16), cb)
                big = jnp.max(m)
                cand = jnp.where(m == big, cb, jnp.int32(2**30))
                mn = jnp.min(cand)
                plsc.store_scatter(
                    idxb,
                    [jnp.full((16,), r, jnp.int32)],
                    cand,
                    mask=cand == mn,
                )
                return c2

            lax.fori_loop(0, _RC, row_body, 0)
            pltpu.sync_copy(idxb, idx_hbm.at[pl.ds(t * _RC, _RC)])
            return carry

        lax.fori_loop(0, n_chunks, chunk_body, 0)

    return _amax_sc


_C = 112
_TOTAL_CHUNKS = -(-_N // _C)
_LAST_OFF = _N - _C


@functools.cache
def _make_gather_sc():
    mesh = plsc.VectorSubcoreMesh(
        core_axis_name="c", subcore_axis_name="s", num_cores=2, num_subcores=16
    )
    nw = mesh.num_cores * mesh.num_subcores

    @functools.partial(
        pl.kernel,
        out_type=jax.ShapeDtypeStruct((_N, _HIDDEN), jnp.float32),
        mesh=mesh,
        scratch_types=[
            pltpu.VMEM((_C,), jnp.int32),
            pltpu.VMEM((_C, _HIDDEN), jnp.float32),
            pltpu.SemaphoreType.DMA,
        ],
    )
    def _gather_sc(table_hbm, idx_hbm, out_hbm, idx_v, rows_v, sem):
        wid = lax.axis_index("s") * mesh.num_cores + lax.axis_index("c")
        n_chunks = (_TOTAL_CHUNKS - wid + nw - 1) // nw

        def body(i, carry):
            t = wid + i * nw
            off = jnp.minimum(t * _C, _LAST_OFF)
            off = pl.multiple_of(off, 8)
            pltpu.sync_copy(idx_hbm.at[pl.ds(off, _C)], idx_v)
            pltpu.async_copy(table_hbm.at[idx_v], rows_v, sem).wait()
            pltpu.sync_copy(rows_v, out_hbm.at[pl.ds(off, _C)])
            return carry

        lax.fori_loop(0, n_chunks, body, 0)

    return _gather_sc


def kernel(clique_attr, emb_weight):
    idx_sc = _make_argmax_sc()(clique_attr)
    idx_tc = _argmax_tc(clique_attr, _SC_BASE // _ROWS_PER_BLK).reshape(_SC_BASE)
    idx = jnp.concatenate([idx_tc, idx_sc])
    return _make_gather_sc()(emb_weight, idx)


# speedup vs baseline: 1.3132x; 1.1710x over previous
import functools

import jax
import jax.numpy as jnp
from jax import lax
from jax.experimental import pallas as pl
from jax.experimental.pallas import tpu as pltpu
from jax.experimental.pallas import tpu_sc as plsc

_N = 50000
_VOCAB = 1000
_HIDDEN = 128


_ROWS_PER_BLK = 5000
_NBLK = _N // _ROWS_PER_BLK


def _argmax_body(x_ref, o_ref):
    x = x_ref[...]
    m = jnp.max(x, axis=1, keepdims=True)
    cols = lax.broadcasted_iota(jnp.int32, x.shape, 1)
    idx = jnp.min(jnp.where(x == m, cols, _VOCAB), axis=1)
    o_ref[...] = idx.reshape(1, 1, _ROWS_PER_BLK)


def _argmax_tc(x):
    return pl.pallas_call(
        _argmax_body,
        grid=(_NBLK,),
        in_specs=[pl.BlockSpec((_ROWS_PER_BLK, _VOCAB), lambda i: (i, 0))],
        out_specs=pl.BlockSpec((1, 1, _ROWS_PER_BLK), lambda i: (i, 0, 0)),
        out_shape=jax.ShapeDtypeStruct((_NBLK, 1, _ROWS_PER_BLK), jnp.int32),
        compiler_params=pltpu.CompilerParams(
            dimension_semantics=("arbitrary",),
        ),
    )(x)


_C = 128
_TOTAL_CHUNKS = -(-_N // _C)
_LAST_OFF = _N - _C


@functools.cache
def _make_gather_sc():
    mesh = plsc.VectorSubcoreMesh(
        core_axis_name="c", subcore_axis_name="s", num_cores=2, num_subcores=16
    )
    nw = mesh.num_cores * mesh.num_subcores

    @functools.partial(
        pl.kernel,
        out_type=jax.ShapeDtypeStruct((_N, _HIDDEN), jnp.float32),
        mesh=mesh,
        scratch_types=[
            pltpu.VMEM((_C,), jnp.int32),
            pltpu.VMEM((_C, _HIDDEN), jnp.float32),
            pltpu.SemaphoreType.DMA,
        ],
    )
    def _gather_sc(table_hbm, idx_hbm, out_hbm, idx_v, rows_v, sem):
        wid = lax.axis_index("s") * mesh.num_cores + lax.axis_index("c")
        n_chunks = (_TOTAL_CHUNKS - wid + nw - 1) // nw

        def body(i, carry):
            t = wid + i * nw
            off = jnp.minimum(t * _C, _LAST_OFF)
            off = pl.multiple_of(off, 8)
            pltpu.sync_copy(idx_hbm.at[pl.ds(off, _C)], idx_v)
            pltpu.async_copy(table_hbm.at[idx_v], rows_v, sem).wait()
            pltpu.sync_copy(rows_v, out_hbm.at[pl.ds(off, _C)])
            return carry

        lax.fori_loop(0, n_chunks, body, 0)

    return _gather_sc


def kernel(clique_attr, emb_weight):
    idx = _argmax_tc(clique_attr).reshape(_N)
    return _make_gather_sc()(emb_weight, idx)
